# Initial kernel scaffold; baseline (speedup 1.0000x reference)
import jax
import jax.numpy as jnp
from jax.experimental import pallas as pl


def kernel(x, W1, b1, W2, b2, Wc, W3, b3, W4, b4):
    raise NotImplementedError("write your pallas kernel here")


# speedup vs baseline: 43.2820x; 43.2820x over previous
import numpy as np
import jax
import jax.numpy as jnp
from jax.experimental import pallas as pl
from jax.experimental.pallas import tpu as pltpu

_INPUT_DIM = 768
_N_HDIM = 2048
_QDIM = 1024
_N_EMBD = 256
_TOPK = _QDIM // 2
_BM = 256


def _body(x_ref, w1t, b1r, w2t, b2r, wct, w3t, b3r, w4t, b4r, ut,
          recon_ref, logits_ref):
    x = x_ref[...]
    h = jnp.dot(x, w1t[...], preferred_element_type=jnp.float32)
    h = jnp.maximum(h + b1r[...], 0.0)
    logits = jnp.dot(h, w2t[...], preferred_element_type=jnp.float32) + b2r[...]
    logits_ref[...] = logits

    bits = jax.lax.bitcast_convert_type(logits, jnp.int32)
    skey = bits ^ (jnp.int32(0x7FFFFFFF) & (bits >> 31))
    ukey = skey ^ jnp.int32(-2 ** 31)

    bm = x.shape[0]
    remaining = jnp.full((bm, 1), float(_TOPK), dtype=jnp.float32)
    prefix = jnp.zeros((bm, 1), dtype=jnp.int32)
    for bit in range(31, -1, -1):
        bmask = np.int32(-2 ** 31) if bit == 31 else np.int32(1 << bit)
        hm = np.int32(-(1 << bit)) if bit < 31 else np.int32(-2 ** 31)
        ones = jnp.where((ukey & hm) == prefix + bmask, 1.0, 0.0)
        c1 = jnp.sum(ones, axis=1, keepdims=True)
        takef = jnp.where(c1 >= remaining, 1.0, 0.0)
        prefix = prefix + takef.astype(jnp.int32) * bmask
        remaining = remaining - (1.0 - takef) * c1

    t_skey = prefix ^ jnp.int32(-2 ** 31)
    gtf = jnp.where(skey > t_skey, 1.0, 0.0)
    eqf = jnp.where(skey == t_skey, 1.0, 0.0)
    need = float(_TOPK) - jnp.sum(gtf, axis=1, keepdims=True)
    excl = jnp.dot(eqf, ut[...], preferred_element_type=jnp.float32)
    maskf = gtf + eqf * jnp.where(excl < need, 1.0, 0.0)

    q = jnp.dot(maskf, wct[...], preferred_element_type=jnp.float32)
    d = jnp.maximum(jnp.dot(q, w3t[...], preferred_element_type=jnp.float32)
                    + b3r[...], 0.0)
    recon_ref[...] = (jnp.dot(d, w4t[...], preferred_element_type=jnp.float32)
                      + b4r[...])


def kernel(x, W1, b1, W2, b2, Wc, W3, b3, W4, b4):
    B = x.shape[0]
    grid = B // _BM
    ut = jnp.asarray(np.triu(np.ones((_QDIM, _QDIM), np.float32), 1))
    args = (x, W1.T, b1.reshape(1, -1), W2.T, b2.reshape(1, -1), Wc.T,
            W3.T, b3.reshape(1, -1), W4.T, b4.reshape(1, -1), ut)
    recon, logits = pl.pallas_call(
        _body,
        grid=(grid,),
        in_specs=[
            pl.BlockSpec((_BM, _INPUT_DIM), lambda i: (i, 0)),
            pl.BlockSpec((_INPUT_DIM, _N_HDIM), lambda i: (0, 0)),
            pl.BlockSpec((1, _N_HDIM), lambda i: (0, 0)),
            pl.BlockSpec((_N_HDIM, _QDIM), lambda i: (0, 0)),
            pl.BlockSpec((1, _QDIM), lambda i: (0, 0)),
            pl.BlockSpec((_QDIM, _N_EMBD), lambda i: (0, 0)),
            pl.BlockSpec((_N_EMBD, _N_HDIM), lambda i: (0, 0)),
            pl.BlockSpec((1, _N_HDIM), lambda i: (0, 0)),
            pl.BlockSpec((_N_HDIM, _INPUT_DIM), lambda i: (0, 0)),
            pl.BlockSpec((1, _INPUT_DIM), lambda i: (0, 0)),
            pl.BlockSpec((_QDIM, _QDIM), lambda i: (0, 0)),
        ],
        out_specs=[
            pl.BlockSpec((_BM, _INPUT_DIM), lambda i: (i, 0)),
            pl.BlockSpec((_BM, _QDIM), lambda i: (i, 0)),
        ],
        out_shape=[
            jax.ShapeDtypeStruct((B, _INPUT_DIM), jnp.float32),
            jax.ShapeDtypeStruct((B, _QDIM), jnp.float32),
        ],
        compiler_params=pltpu.CompilerParams(
            dimension_semantics=("arbitrary",)),
    )(*args)
    return (recon, logits, jnp.float32(0.0))


# speedup vs baseline: 44.3918x; 1.0256x over previous
import numpy as np
import jax
import jax.numpy as jnp
from jax.experimental import pallas as pl
from jax.experimental.pallas import tpu as pltpu

_INPUT_DIM = 768
_N_HDIM = 2048
_QDIM = 1024
_N_EMBD = 256
_TOPK = _QDIM // 2
_BM = 512


def _body(x_ref, w1t, b1r, w2t, b2r, wct, w3t, b3r, w4t, b4r, ut,
          recon_ref, logits_ref):
    x = x_ref[...]
    h = jnp.dot(x, w1t[...], preferred_element_type=jnp.float32)
    h = jnp.maximum(h + b1r[...], 0.0)
    logits = jnp.dot(h, w2t[...], preferred_element_type=jnp.float32) + b2r[...]
    logits_ref[...] = logits

    bits = jax.lax.bitcast_convert_type(logits, jnp.int32)
    skey = bits ^ (jnp.int32(0x7FFFFFFF) & (bits >> 31))
    ukey = skey ^ jnp.int32(-2 ** 31)

    bm = x.shape[0]
    remaining = jnp.full((bm, 1), float(_TOPK), dtype=jnp.float32)
    prefix = jnp.zeros((bm, 1), dtype=jnp.int32)
    for bit in range(31, -1, -1):
        bmask = np.int32(-2 ** 31) if bit == 31 else np.int32(1 << bit)
        hm = np.int32(-(1 << bit)) if bit < 31 else np.int32(-2 ** 31)
        ones = jnp.where((ukey & hm) == prefix + bmask, 1.0, 0.0)
        c1 = jnp.sum(ones, axis=1, keepdims=True)
        takef = jnp.where(c1 >= remaining, 1.0, 0.0)
        prefix = prefix + takef.astype(jnp.int32) * bmask
        remaining = remaining - (1.0 - takef) * c1

    t_skey = prefix ^ jnp.int32(-2 ** 31)
    gtf = jnp.where(skey > t_skey, 1.0, 0.0)
    eqf = jnp.where(skey == t_skey, 1.0, 0.0)
    need = float(_TOPK) - jnp.sum(gtf, axis=1, keepdims=True)
    excl = jnp.dot(eqf, ut[...], preferred_element_type=jnp.float32)
    maskf = gtf + eqf * jnp.where(excl < need, 1.0, 0.0)

    q = jnp.dot(maskf, wct[...], preferred_element_type=jnp.float32)
    d = jnp.maximum(jnp.dot(q, w3t[...], preferred_element_type=jnp.float32)
                    + b3r[...], 0.0)
    recon_ref[...] = (jnp.dot(d, w4t[...], preferred_element_type=jnp.float32)
                      + b4r[...])


def kernel(x, W1, b1, W2, b2, Wc, W3, b3, W4, b4):
    B = x.shape[0]
    grid = B // _BM
    ut = jnp.asarray(np.triu(np.ones((_QDIM, _QDIM), np.float32), 1))
    args = (x, W1.T, b1.reshape(1, -1), W2.T, b2.reshape(1, -1), Wc.T,
            W3.T, b3.reshape(1, -1), W4.T, b4.reshape(1, -1), ut)
    recon, logits = pl.pallas_call(
        _body,
        grid=(grid,),
        in_specs=[
            pl.BlockSpec((_BM, _INPUT_DIM), lambda i: (i, 0)),
            pl.BlockSpec((_INPUT_DIM, _N_HDIM), lambda i: (0, 0)),
            pl.BlockSpec((1, _N_HDIM), lambda i: (0, 0)),
            pl.BlockSpec((_N_HDIM, _QDIM), lambda i: (0, 0)),
            pl.BlockSpec((1, _QDIM), lambda i: (0, 0)),
            pl.BlockSpec((_QDIM, _N_EMBD), lambda i: (0, 0)),
            pl.BlockSpec((_N_EMBD, _N_HDIM), lambda i: (0, 0)),
            pl.BlockSpec((1, _N_HDIM), lambda i: (0, 0)),
            pl.BlockSpec((_N_HDIM, _INPUT_DIM), lambda i: (0, 0)),
            pl.BlockSpec((1, _INPUT_DIM), lambda i: (0, 0)),
            pl.BlockSpec((_QDIM, _QDIM), lambda i: (0, 0)),
        ],
        out_specs=[
            pl.BlockSpec((_BM, _INPUT_DIM), lambda i: (i, 0)),
            pl.BlockSpec((_BM, _QDIM), lambda i: (i, 0)),
        ],
        out_shape=[
            jax.ShapeDtypeStruct((B, _INPUT_DIM), jnp.float32),
            jax.ShapeDtypeStruct((B, _QDIM), jnp.float32),
        ],
        compiler_params=pltpu.CompilerParams(
            dimension_semantics=("arbitrary",)),
    )(*args)
    return (recon, logits, jnp.float32(0.0))


# speedup vs baseline: 51.2875x; 1.1553x over previous
import numpy as np
import jax
import jax.numpy as jnp
from jax.experimental import pallas as pl
from jax.experimental.pallas import tpu as pltpu

_INPUT_DIM = 768
_N_HDIM = 2048
_QDIM = 1024
_N_EMBD = 256
_TOPK = _QDIM // 2
_BM = 512


def _body(x_ref, w1t, b1r, w2t, b2r, wct, w3t, b3r, w4t, b4r, ut,
          recon_ref, logits_ref):
    x = x_ref[...]
    h = jnp.dot(x, w1t[...], preferred_element_type=jnp.float32)
    h = jnp.maximum(h + b1r[...], 0.0)
    logits = jnp.dot(h, w2t[...], preferred_element_type=jnp.float32) + b2r[...]
    logits_ref[...] = logits

    bits = jax.lax.bitcast_convert_type(logits, jnp.int32)
    skey = bits ^ (jnp.int32(0x7FFFFFFF) & (bits >> 31))

    bm = x.shape[0]
    prefix = jnp.zeros((bm, 1), dtype=jnp.int32)
    for bit in range(31, -1, -1):
        bmask = np.int32(-2 ** 31) if bit == 31 else np.int32(1 << bit)
        ts = (prefix + bmask) ^ jnp.int32(-2 ** 31)
        cnt = jnp.sum(jnp.where(skey >= ts, 1.0, 0.0), axis=1, keepdims=True)
        keep = jnp.where(cnt >= float(_TOPK), 1, 0).astype(jnp.int32)
        prefix = prefix + keep * bmask

    t_skey = prefix ^ jnp.int32(-2 ** 31)
    gtf = jnp.where(skey > t_skey, 1.0, 0.0)
    eqf = jnp.where(skey == t_skey, 1.0, 0.0)
    need = float(_TOPK) - jnp.sum(gtf, axis=1, keepdims=True)
    excl = jnp.dot(eqf, ut[...], preferred_element_type=jnp.float32)
    maskf = gtf + eqf * jnp.where(excl < need, 1.0, 0.0)

    q = jnp.dot(maskf, wct[...], preferred_element_type=jnp.float32)
    d = jnp.maximum(jnp.dot(q, w3t[...], preferred_element_type=jnp.float32)
                    + b3r[...], 0.0)
    recon_ref[...] = (jnp.dot(d, w4t[...], preferred_element_type=jnp.float32)
                      + b4r[...])


def kernel(x, W1, b1, W2, b2, Wc, W3, b3, W4, b4):
    B = x.shape[0]
    grid = B // _BM
    ut = jnp.asarray(np.triu(np.ones((_QDIM, _QDIM), np.float32), 1))
    args = (x, W1.T, b1.reshape(1, -1), W2.T, b2.reshape(1, -1), Wc.T,
            W3.T, b3.reshape(1, -1), W4.T, b4.reshape(1, -1), ut)
    recon, logits = pl.pallas_call(
        _body,
        grid=(grid,),
        in_specs=[
            pl.BlockSpec((_BM, _INPUT_DIM), lambda i: (i, 0)),
            pl.BlockSpec((_INPUT_DIM, _N_HDIM), lambda i: (0, 0)),
            pl.BlockSpec((1, _N_HDIM), lambda i: (0, 0)),
            pl.BlockSpec((_N_HDIM, _QDIM), lambda i: (0, 0)),
            pl.BlockSpec((1, _QDIM), lambda i: (0, 0)),
            pl.BlockSpec((_QDIM, _N_EMBD), lambda i: (0, 0)),
            pl.BlockSpec((_N_EMBD, _N_HDIM), lambda i: (0, 0)),
            pl.BlockSpec((1, _N_HDIM), lambda i: (0, 0)),
            pl.BlockSpec((_N_HDIM, _INPUT_DIM), lambda i: (0, 0)),
            pl.BlockSpec((1, _INPUT_DIM), lambda i: (0, 0)),
            pl.BlockSpec((_QDIM, _QDIM), lambda i: (0, 0)),
        ],
        out_specs=[
            pl.BlockSpec((_BM, _INPUT_DIM), lambda i: (i, 0)),
            pl.BlockSpec((_BM, _QDIM), lambda i: (i, 0)),
        ],
        out_shape=[
            jax.ShapeDtypeStruct((B, _INPUT_DIM), jnp.float32),
            jax.ShapeDtypeStruct((B, _QDIM), jnp.float32),
        ],
        compiler_params=pltpu.CompilerParams(
            dimension_semantics=("arbitrary",)),
    )(*args)
    return (recon, logits, jnp.float32(0.0))


# speedup vs baseline: 52.6353x; 1.0263x over previous
import numpy as np
import jax
import jax.numpy as jnp
from jax.experimental import pallas as pl
from jax.experimental.pallas import tpu as pltpu

_INPUT_DIM = 768
_N_HDIM = 2048
_QDIM = 1024
_N_EMBD = 256
_TOPK = _QDIM // 2
_BM = 512


def _body(x_ref, w1t, b1r, w2t, b2r, wct, w3t, b3r, w4t, b4r, ut,
          recon_ref, logits_ref):
    x = x_ref[...]
    h = jnp.dot(x, w1t[...], preferred_element_type=jnp.float32)
    h = jnp.maximum(h + b1r[...], 0.0)
    logits = jnp.dot(h, w2t[...], preferred_element_type=jnp.float32) + b2r[...]
    logits_ref[...] = logits

    bits = jax.lax.bitcast_convert_type(logits, jnp.int32)
    skey = bits ^ (jnp.int32(0x7FFFFFFF) & (bits >> 31))

    bm = x.shape[0]
    prefix = jnp.zeros((bm, 1), dtype=jnp.int32)
    for bit in range(31, -1, -1):
        bmask = np.int32(-2 ** 31) if bit == 31 else np.int32(1 << bit)
        ts = (prefix + bmask) ^ jnp.int32(-2 ** 31)
        cnt = jnp.sum(jnp.where(skey >= ts, 1.0, 0.0), axis=1, keepdims=True)
        keep = jnp.where(cnt >= float(_TOPK), 1, 0).astype(jnp.int32)
        prefix = prefix + keep * bmask

    t_skey = prefix ^ jnp.int32(-2 ** 31)
    gtf = jnp.where(skey > t_skey, 1.0, 0.0)
    eqf = jnp.where(skey == t_skey, 1.0, 0.0)
    need = float(_TOPK) - jnp.sum(gtf, axis=1, keepdims=True)
    excl = jnp.dot(eqf.astype(jnp.bfloat16), ut[...],
                   preferred_element_type=jnp.float32)
    maskf = gtf + eqf * jnp.where(excl < need, 1.0, 0.0)

    q = jnp.dot(maskf.astype(jnp.bfloat16), wct[...],
                preferred_element_type=jnp.float32)
    d = jnp.maximum(jnp.dot(q.astype(jnp.bfloat16), w3t[...],
                            preferred_element_type=jnp.float32) + b3r[...], 0.0)
    recon_ref[...] = (jnp.dot(d.astype(jnp.bfloat16), w4t[...],
                              preferred_element_type=jnp.float32) + b4r[...])


def kernel(x, W1, b1, W2, b2, Wc, W3, b3, W4, b4):
    B = x.shape[0]
    grid = B // _BM
    ut = jnp.asarray(np.triu(np.ones((_QDIM, _QDIM), np.float32), 1),
                     dtype=jnp.bfloat16)
    args = (x, W1.T, b1.reshape(1, -1), W2.T, b2.reshape(1, -1),
            Wc.T.astype(jnp.bfloat16), W3.T.astype(jnp.bfloat16),
            b3.reshape(1, -1), W4.T.astype(jnp.bfloat16),
            b4.reshape(1, -1), ut)
    recon, logits = pl.pallas_call(
        _body,
        grid=(grid,),
        in_specs=[
            pl.BlockSpec((_BM, _INPUT_DIM), lambda i: (i, 0)),
            pl.BlockSpec((_INPUT_DIM, _N_HDIM), lambda i: (0, 0)),
            pl.BlockSpec((1, _N_HDIM), lambda i: (0, 0)),
            pl.BlockSpec((_N_HDIM, _QDIM), lambda i: (0, 0)),
            pl.BlockSpec((1, _QDIM), lambda i: (0, 0)),
            pl.BlockSpec((_QDIM, _N_EMBD), lambda i: (0, 0)),
            pl.BlockSpec((_N_EMBD, _N_HDIM), lambda i: (0, 0)),
            pl.BlockSpec((1, _N_HDIM), lambda i: (0, 0)),
            pl.BlockSpec((_N_HDIM, _INPUT_DIM), lambda i: (0, 0)),
            pl.BlockSpec((1, _INPUT_DIM), lambda i: (0, 0)),
            pl.BlockSpec((_QDIM, _QDIM), lambda i: (0, 0)),
        ],
        out_specs=[
            pl.BlockSpec((_BM, _INPUT_DIM), lambda i: (i, 0)),
            pl.BlockSpec((_BM, _QDIM), lambda i: (i, 0)),
        ],
        out_shape=[
            jax.ShapeDtypeStruct((B, _INPUT_DIM), jnp.float32),
            jax.ShapeDtypeStruct((B, _QDIM), jnp.float32),
        ],
        compiler_params=pltpu.CompilerParams(
            dimension_semantics=("arbitrary",)),
    )(*args)
    return (recon, logits, jnp.float32(0.0))


# speedup vs baseline: 58.3195x; 1.1080x over previous
import numpy as np
import jax
import jax.numpy as jnp
from jax.experimental import pallas as pl
from jax.experimental.pallas import tpu as pltpu

_INPUT_DIM = 768
_N_HDIM = 2048
_QDIM = 1024
_N_EMBD = 256
_TOPK = _QDIM // 2
_BM = 512


_DN_RT = (((1,), (1,)), ((), ()))


def _body(x_ref, w1r, b1r, w2r, b2r, wcr, w3r, b3r, w4r, b4r, ut,
          recon_ref, logits_ref):
    x = x_ref[...]
    h = jax.lax.dot_general(x, w1r[...], _DN_RT,
                            preferred_element_type=jnp.float32)
    h = jnp.maximum(h + b1r[...], 0.0)
    logits = jax.lax.dot_general(h, w2r[...], _DN_RT,
                                 preferred_element_type=jnp.float32) + b2r[...]
    logits_ref[...] = logits

    bits = jax.lax.bitcast_convert_type(logits, jnp.int32)
    skey = bits ^ (jnp.int32(0x7FFFFFFF) & (bits >> 31))

    bm = x.shape[0]
    prefix = jnp.zeros((bm, 1), dtype=jnp.int32)
    for bit in range(31, -1, -1):
        bmask = np.int32(-2 ** 31) if bit == 31 else np.int32(1 << bit)
        ts = (prefix + bmask) ^ jnp.int32(-2 ** 31)
        cnt = jnp.sum(jnp.where(skey >= ts, 1.0, 0.0), axis=1, keepdims=True)
        keep = jnp.where(cnt >= float(_TOPK), 1, 0).astype(jnp.int32)
        prefix = prefix + keep * bmask

    t_skey = prefix ^ jnp.int32(-2 ** 31)
    gtf = jnp.where(skey > t_skey, 1.0, 0.0)
    eqf = jnp.where(skey == t_skey, 1.0, 0.0)
    need = float(_TOPK) - jnp.sum(gtf, axis=1, keepdims=True)
    excl = jnp.dot(eqf.astype(jnp.bfloat16), ut[...],
                   preferred_element_type=jnp.float32)
    maskf = gtf + eqf * jnp.where(excl < need, 1.0, 0.0)

    q = jax.lax.dot_general(maskf.astype(jnp.bfloat16), wcr[...], _DN_RT,
                            preferred_element_type=jnp.float32)
    d = jnp.maximum(
        jax.lax.dot_general(q.astype(jnp.bfloat16), w3r[...], _DN_RT,
                            preferred_element_type=jnp.float32) + b3r[...], 0.0)
    recon_ref[...] = (
        jax.lax.dot_general(d.astype(jnp.bfloat16), w4r[...], _DN_RT,
                            preferred_element_type=jnp.float32) + b4r[...])


def kernel(x, W1, b1, W2, b2, Wc, W3, b3, W4, b4):
    B = x.shape[0]
    grid = B // _BM
    ut = jnp.asarray(np.triu(np.ones((_QDIM, _QDIM), np.float32), 1),
                     dtype=jnp.bfloat16)
    args = (x, W1, b1.reshape(1, -1), W2, b2.reshape(1, -1),
            Wc.astype(jnp.bfloat16), W3.astype(jnp.bfloat16),
            b3.reshape(1, -1), W4.astype(jnp.bfloat16),
            b4.reshape(1, -1), ut)
    recon, logits = pl.pallas_call(
        _body,
        grid=(grid,),
        in_specs=[
            pl.BlockSpec((_BM, _INPUT_DIM), lambda i: (i, 0)),
            pl.BlockSpec((_N_HDIM, _INPUT_DIM), lambda i: (0, 0)),
            pl.BlockSpec((1, _N_HDIM), lambda i: (0, 0)),
            pl.BlockSpec((_QDIM, _N_HDIM), lambda i: (0, 0)),
            pl.BlockSpec((1, _QDIM), lambda i: (0, 0)),
            pl.BlockSpec((_N_EMBD, _QDIM), lambda i: (0, 0)),
            pl.BlockSpec((_N_HDIM, _N_EMBD), lambda i: (0, 0)),
            pl.BlockSpec((1, _N_HDIM), lambda i: (0, 0)),
            pl.BlockSpec((_INPUT_DIM, _N_HDIM), lambda i: (0, 0)),
            pl.BlockSpec((1, _INPUT_DIM), lambda i: (0, 0)),
            pl.BlockSpec((_QDIM, _QDIM), lambda i: (0, 0)),
        ],
        out_specs=[
            pl.BlockSpec((_BM, _INPUT_DIM), lambda i: (i, 0)),
            pl.BlockSpec((_BM, _QDIM), lambda i: (i, 0)),
        ],
        out_shape=[
            jax.ShapeDtypeStruct((B, _INPUT_DIM), jnp.float32),
            jax.ShapeDtypeStruct((B, _QDIM), jnp.float32),
        ],
        compiler_params=pltpu.CompilerParams(
            dimension_semantics=("arbitrary",)),
    )(*args)
    return (recon, logits, jnp.float32(0.0))
